# Initial kernel scaffold; baseline (speedup 1.0000x reference)
import jax
import jax.numpy as jnp
from jax.experimental import pallas as pl


def kernel(y, eval_gene_idx, train_highly_gene_idx, index, unnorm_index, thresh):
    raise NotImplementedError("write your pallas kernel here")


# speedup vs baseline: 3.5918x; 3.5918x over previous
import jax
import jax.numpy as jnp
from jax import lax
from jax.experimental import pallas as pl
from jax.experimental.pallas import tpu as pltpu
from jax.experimental.pallas import tpu_sc as plsc

N_AUX = 500000
N_TOT = 1000000
NW = 32
CH_A = 15632
CH_A_TAIL = N_AUX - (NW - 1) * CH_A
CH_B = 20000
NCH_B = N_AUX // CH_B
FILL = 20000
NFILL = N_TOT // FILL
BIG = 2 ** 30


def _vmin(v):
    iota = lax.iota(jnp.int32, 16)
    for s in (8, 4, 2, 1):
        v = jnp.minimum(v, v.at[iota ^ s].get(mode="promise_in_bounds"))
    return v


def _mesh():
    return plsc.VectorSubcoreMesh(core_axis_name="c", subcore_axis_name="s",
                                  num_cores=2, num_subcores=16)


def _stage1(index_hbm, t16_hbm, mask_hbm, pdiff_hbm, ppos_hbm,
            xch, mch, tv, rdv, rpv):
    wid = lax.axis_index("c") * 16 + lax.axis_index("s")
    pltpu.sync_copy(t16_hbm, tv)
    t = tv[...]
    iota = lax.iota(jnp.int32, 16)

    def work(n):
        base = wid * CH_A
        pltpu.sync_copy(index_hbm.at[pl.ds(base, n)], xch.at[pl.ds(0, n)])

        def body(j, carry):
            bestd, bestp = carry
            x = xch[pl.ds(j * 16, 16)]
            m = 1.0 / (1.0 + jnp.exp((x - t) * 10.0))
            mch[pl.ds(j * 16, 16)] = m
            d = jnp.abs(x - t)
            pos = base + j * 16 + iota
            take = d < bestd
            return (jnp.where(take, d, bestd), jnp.where(take, pos, bestp))

        bestd, bestp = lax.fori_loop(
            0, n // 16, body,
            (jnp.full((16,), jnp.inf, jnp.float32), jnp.zeros((16,), jnp.int32)))
        dmin = _vmin(bestd)
        pmin = _vmin(jnp.where(bestd == dmin, bestp, jnp.int32(BIG)))
        rdv[...] = dmin
        rpv[...] = pmin
        pltpu.sync_copy(mch.at[pl.ds(0, n)], mask_hbm.at[pl.ds(base, n)])
        pltpu.sync_copy(rdv, pdiff_hbm.at[wid])
        pltpu.sync_copy(rpv, ppos_hbm.at[wid])

    @pl.when(wid < NW - 1)
    def _():
        work(CH_A)

    @pl.when(wid == NW - 1)
    def _():
        work(CH_A_TAIL)


def _stage2(tidx_hbm, mask_hbm, pdiff_hbm, ppos_hbm, un_hbm,
            w_hbm, k_hbm,
            ones_v, ib0, ib1, vb0, vb1, pdv, ppv, posv, kv,
            sem_fill, sem_g, sem_s, sem_k):
    wid = lax.axis_index("c") * 16 + lax.axis_index("s")

    @pl.when(wid == 0)
    def _():
        ibs = (ib0, ib1)
        vbs = (vb0, vb1)

        def start_gather(c, b):
            d1 = pltpu.async_copy(tidx_hbm.at[pl.ds(c * CH_B, CH_B)], ibs[b], sem_g)
            d2 = pltpu.async_copy(mask_hbm.at[pl.ds(c * CH_B, CH_B)], vbs[b], sem_g)
            return (d1, d2)

        pending = start_gather(0, 0)

        def fb(i, z):
            ones_v[pl.ds(i * 16, 16)] = jnp.full((16,), 1.0, jnp.float32)
            return z

        lax.fori_loop(0, FILL // 16, fb, 0)
        fills = [pltpu.async_copy(ones_v, w_hbm.at[pl.ds(i * FILL, FILL)], sem_fill)
                 for i in range(NFILL)]
        for d in fills:
            d.wait()

        for c in range(NCH_B):
            b = c % 2
            pending[0].wait()
            pending[1].wait()
            if c + 1 < NCH_B:
                pending = start_gather(c + 1, 1 - b)
            pltpu.async_copy(vbs[b], w_hbm.at[ibs[b]], sem_s).wait()

        pltpu.sync_copy(pdiff_hbm, pdv)
        pltpu.sync_copy(ppos_hbm, ppv)

        def rb(r, carry):
            bd, bp = carry
            d = pdv[r][0]
            p = ppv[r][0]
            take = (d < bd) | ((d == bd) & (p < bp))
            return (jnp.where(take, d, bd), jnp.where(take, p, bp))

        bd, bp = lax.fori_loop(0, NW, rb,
                               (jnp.float32(jnp.inf), jnp.int32(BIG)))
        posv[...] = jnp.full((16,), bp, jnp.int32)
        pltpu.async_copy(un_hbm.at[posv], kv, sem_k).wait()
        pltpu.sync_copy(kv, k_hbm)


def kernel(y, eval_gene_idx, train_highly_gene_idx, index, unnorm_index, thresh):
    t16 = jnp.broadcast_to(jnp.asarray(thresh, jnp.float32), (16,))

    k1 = pl.kernel(
        _stage1,
        out_type=(
            jax.ShapeDtypeStruct((N_AUX,), jnp.float32),
            jax.ShapeDtypeStruct((NW, 16), jnp.float32),
            jax.ShapeDtypeStruct((NW, 16), jnp.int32),
        ),
        mesh=_mesh(),
        scratch_types=(
            pltpu.VMEM((CH_A,), jnp.float32),
            pltpu.VMEM((CH_A,), jnp.float32),
            pltpu.VMEM((16,), jnp.float32),
            pltpu.VMEM((16,), jnp.float32),
            pltpu.VMEM((16,), jnp.int32),
        ),
    )
    mask, pdiff, ppos = k1(index, t16)

    k2 = pl.kernel(
        _stage2,
        out_type=(
            jax.ShapeDtypeStruct((N_TOT,), jnp.float32),
            jax.ShapeDtypeStruct((16,), jnp.int32),
        ),
        mesh=_mesh(),
        scratch_types=(
            pltpu.VMEM((FILL,), jnp.float32),
            pltpu.VMEM((CH_B,), jnp.int32),
            pltpu.VMEM((CH_B,), jnp.int32),
            pltpu.VMEM((CH_B,), jnp.float32),
            pltpu.VMEM((CH_B,), jnp.float32),
            pltpu.VMEM((NW, 16), jnp.float32),
            pltpu.VMEM((NW, 16), jnp.int32),
            pltpu.VMEM((16,), jnp.int32),
            pltpu.VMEM((16,), jnp.int32),
            pltpu.SemaphoreType.DMA,
            pltpu.SemaphoreType.DMA,
            pltpu.SemaphoreType.DMA,
            pltpu.SemaphoreType.DMA,
        ),
    )
    w, k16 = k2(train_highly_gene_idx, mask, pdiff, ppos, unnorm_index)
    return (w, w, thresh, k16[0])


# speedup vs baseline: 11.3043x; 3.1473x over previous
import jax
import jax.numpy as jnp
from jax import lax
from jax.experimental import pallas as pl
from jax.experimental.pallas import tpu as pltpu
from jax.experimental.pallas import tpu_sc as plsc

N_AUX = 500000
N_TOT = 1000000
NW = 32
CH_A = 15632
CH_A_TAIL = N_AUX - (NW - 1) * CH_A
CH_B = 10000
NCH_B = N_AUX // CH_B
FILL = 15632
SLICE = 4 * FILL
SLICE_T = N_TOT - 15 * SLICE
BIG = 2 ** 30


def _vmin(v):
    iota = lax.iota(jnp.int32, 16)
    for s in (8, 4, 2, 1):
        v = jnp.minimum(v, v.at[iota ^ s].get(mode="promise_in_bounds"))
    return v


def _mesh():
    return plsc.VectorSubcoreMesh(core_axis_name="c", subcore_axis_name="s",
                                  num_cores=2, num_subcores=16)


def _stage1(index_hbm, t16_hbm, mask_hbm, pdiff_hbm, ppos_hbm,
            xch, mch, tv, rdv, rpv):
    wid = lax.axis_index("c") * 16 + lax.axis_index("s")
    pltpu.sync_copy(t16_hbm, tv)
    t = tv[...]
    iota = lax.iota(jnp.int32, 16)

    def work(n):
        base = wid * CH_A
        pltpu.sync_copy(index_hbm.at[pl.ds(base, n)], xch.at[pl.ds(0, n)])

        def body(j, carry):
            bestd, bestp = carry
            x = xch[pl.ds(j * 16, 16)]
            m = 1.0 / (1.0 + jnp.exp((x - t) * 10.0))
            mch[pl.ds(j * 16, 16)] = m
            d = jnp.abs(x - t)
            pos = base + j * 16 + iota
            take = d < bestd
            return (jnp.where(take, d, bestd), jnp.where(take, pos, bestp))

        bestd, bestp = lax.fori_loop(
            0, n // 16, body,
            (jnp.full((16,), jnp.inf, jnp.float32), jnp.zeros((16,), jnp.int32)))
        dmin = _vmin(bestd)
        pmin = _vmin(jnp.where(bestd == dmin, bestp, jnp.int32(BIG)))
        rdv[...] = dmin
        rpv[...] = pmin
        pltpu.sync_copy(mch.at[pl.ds(0, n)], mask_hbm.at[pl.ds(base, n)])
        pltpu.sync_copy(rdv, pdiff_hbm.at[wid])
        pltpu.sync_copy(rpv, ppos_hbm.at[wid])

    @pl.when(wid < NW - 1)
    def _():
        work(CH_A)

    @pl.when(wid == NW - 1)
    def _():
        work(CH_A_TAIL)


def _stage2(tidx_hbm, mask_hbm, pdiff_hbm, ppos_hbm, un_hbm,
            w_hbm, k_hbm,
            wsh, ones_v, ib0, ib1, vb0, vb1, pdv, ppv, posv, kv,
            sem_fill, sem_g, sem_s, sem_k):
    cid = lax.axis_index("c")
    sid = lax.axis_index("s")
    wid = cid * 16 + sid

    @pl.when(cid == 0)
    def _():
        def fb(i, z):
            ones_v[pl.ds(i * 16, 16)] = jnp.full((16,), 1.0, jnp.float32)
            return z

        lax.fori_loop(0, FILL // 16, fb, 0)
        base = sid * SLICE

        @pl.when(sid < 15)
        def _():
            for j in range(SLICE // FILL):
                pltpu.sync_copy(ones_v, wsh.at[pl.ds(base + j * FILL, FILL)])

        @pl.when(sid == 15)
        def _():
            for j in range(SLICE_T // FILL):
                pltpu.sync_copy(ones_v, wsh.at[pl.ds(base + j * FILL, FILL)])
            rem = SLICE_T % FILL
            if rem:
                pltpu.sync_copy(ones_v.at[pl.ds(0, rem)],
                                wsh.at[pl.ds(N_TOT - rem, rem)])

        plsc.subcore_barrier()

    @pl.when(wid == 0)
    def _():
        ibs = (ib0, ib1)
        vbs = (vb0, vb1)

        def start_gather(c, b):
            d1 = pltpu.async_copy(tidx_hbm.at[pl.ds(c * CH_B, CH_B)], ibs[b], sem_g)
            d2 = pltpu.async_copy(mask_hbm.at[pl.ds(c * CH_B, CH_B)], vbs[b], sem_g)
            return (d1, d2)

        pending = start_gather(0, 0)
        for c in range(NCH_B):
            b = c % 2
            pending[0].wait()
            pending[1].wait()
            if c + 1 < NCH_B:
                pending = start_gather(c + 1, 1 - b)
            pltpu.async_copy(vbs[b], wsh.at[ibs[b]], sem_s).wait()

    @pl.when(cid == 0)
    def _():
        plsc.subcore_barrier()
        base = sid * SLICE

        def bounce(off, n):
            pltpu.sync_copy(wsh.at[pl.ds(off, n)], ones_v.at[pl.ds(0, n)])
            pltpu.sync_copy(ones_v.at[pl.ds(0, n)], w_hbm.at[pl.ds(off, n)])

        @pl.when(sid < 15)
        def _():
            for j in range(SLICE // FILL):
                bounce(base + j * FILL, FILL)

        @pl.when(sid == 15)
        def _():
            for j in range(SLICE_T // FILL):
                bounce(base + j * FILL, FILL)
            rem = SLICE_T % FILL
            if rem:
                bounce(N_TOT - rem, rem)

    @pl.when(wid == 0)
    def _():
        pltpu.sync_copy(pdiff_hbm, pdv)
        pltpu.sync_copy(ppos_hbm, ppv)

        def rb(r, carry):
            bd, bp = carry
            d = pdv[r][0]
            p = ppv[r][0]
            take = (d < bd) | ((d == bd) & (p < bp))
            return (jnp.where(take, d, bd), jnp.where(take, p, bp))

        bd, bp = lax.fori_loop(0, NW, rb,
                               (jnp.float32(jnp.inf), jnp.int32(BIG)))
        posv[...] = jnp.full((16,), bp, jnp.int32)
        pltpu.async_copy(un_hbm.at[posv], kv, sem_k).wait()
        pltpu.sync_copy(kv, k_hbm)


def kernel(y, eval_gene_idx, train_highly_gene_idx, index, unnorm_index, thresh):
    t16 = jnp.broadcast_to(jnp.asarray(thresh, jnp.float32), (16,))

    k1 = pl.kernel(
        _stage1,
        out_type=(
            jax.ShapeDtypeStruct((N_AUX,), jnp.float32),
            jax.ShapeDtypeStruct((NW, 16), jnp.float32),
            jax.ShapeDtypeStruct((NW, 16), jnp.int32),
        ),
        mesh=_mesh(),
        scratch_types=(
            pltpu.VMEM((CH_A,), jnp.float32),
            pltpu.VMEM((CH_A,), jnp.float32),
            pltpu.VMEM((16,), jnp.float32),
            pltpu.VMEM((16,), jnp.float32),
            pltpu.VMEM((16,), jnp.int32),
        ),
    )
    mask, pdiff, ppos = k1(index, t16)

    k2 = pl.kernel(
        _stage2,
        out_type=(
            jax.ShapeDtypeStruct((N_TOT,), jnp.float32),
            jax.ShapeDtypeStruct((16,), jnp.int32),
        ),
        mesh=_mesh(),
        scratch_types=(
            pltpu.VMEM_SHARED((N_TOT,), jnp.float32),
            pltpu.VMEM((FILL,), jnp.float32),
            pltpu.VMEM((CH_B,), jnp.int32),
            pltpu.VMEM((CH_B,), jnp.int32),
            pltpu.VMEM((CH_B,), jnp.float32),
            pltpu.VMEM((CH_B,), jnp.float32),
            pltpu.VMEM((NW, 16), jnp.float32),
            pltpu.VMEM((NW, 16), jnp.int32),
            pltpu.VMEM((16,), jnp.int32),
            pltpu.VMEM((16,), jnp.int32),
            pltpu.SemaphoreType.DMA,
            pltpu.SemaphoreType.DMA,
            pltpu.SemaphoreType.DMA,
            pltpu.SemaphoreType.DMA,
        ),
    )
    w, k16 = k2(train_highly_gene_idx, mask, pdiff, ppos, unnorm_index)
    return (w, w, thresh, k16[0])
